# baseline (speedup 1.0000x reference)
import functools

import jax
import jax.numpy as jnp
from jax.experimental import pallas as pl
from jax.experimental.pallas import tpu as pltpu

NEG_SLOPE = 0.2


def _node_kernel(x_ref, w_ref, aw_ref, xp_ref, asd_ref):
    xp = jnp.dot(x_ref[...], w_ref[...], preferred_element_type=jnp.float32)
    xp_ref[...] = xp
    asd_ref[...] = jnp.dot(xp, aw_ref[...], preferred_element_type=jnp.float32)


def _edge_kernel(src_ref, dst_ref, asd_ref, xp_ref, exp_ref,
                 alpha_ref, out_ref, acc_ref, scale_ref, *, n_heads):
    p = pl.program_id(0)
    i = pl.program_id(1)
    blk = src_ref.shape[2]
    k = n_heads

    @pl.when(jnp.logical_and(p == 0, i == 0))
    def _init():
        acc_ref[:, pl.ds(0, k)] = jnp.full(
            (acc_ref.shape[0], k), -jnp.inf, jnp.float32)
        acc_ref[:, pl.ds(k, k)] = jnp.zeros(
            (acc_ref.shape[0], k), jnp.float32)
        out_ref[...] = jnp.zeros_like(out_ref)

    def raw_alpha(j):
        s = src_ref[0, 0, j]
        d = dst_ref[0, 0, j]
        ar = (asd_ref[pl.ds(s, 1), pl.ds(0, k)]
              + asd_ref[pl.ds(d, 1), pl.ds(k, k)])
        ar = jnp.where(ar >= 0, ar, NEG_SLOPE * ar)
        return s, d, ar

    @pl.when(p == 0)
    def _pass_max():
        def body(j, c):
            _, d, ar = raw_alpha(j)
            cur = acc_ref[pl.ds(d, 1), pl.ds(0, k)]
            acc_ref[pl.ds(d, 1), pl.ds(0, k)] = jnp.maximum(cur, ar)
            return c
        jax.lax.fori_loop(0, blk, body, 0)

    @pl.when(p == 1)
    def _pass_sum():
        def body(j, c):
            _, d, ar = raw_alpha(j)
            ae = jnp.exp(ar - acc_ref[pl.ds(d, 1), pl.ds(0, k)])
            acc_ref[pl.ds(d, 1), pl.ds(k, k)] += ae
            return c
        jax.lax.fori_loop(0, blk, body, 0)

    @pl.when(p == 2)
    def _pass_norm_scatter():
        def body_alpha(j, c):
            _, d, ar = raw_alpha(j)
            ae = jnp.exp(ar - acc_ref[pl.ds(d, 1), pl.ds(0, k)])
            al = ae / (acc_ref[pl.ds(d, 1), pl.ds(k, k)] + 1e-16)
            alpha_ref[pl.ds(j, 1), :] = al
            return c
        jax.lax.fori_loop(0, blk, body_alpha, 0)
        scale_ref[...] = jnp.dot(alpha_ref[...], exp_ref[...],
                                 preferred_element_type=jnp.float32)

        def body_msg(j, c):
            s = src_ref[0, 0, j]
            d = dst_ref[0, 0, j]
            out_ref[pl.ds(d, 1), :] += (xp_ref[pl.ds(s, 1), :]
                                        * scale_ref[pl.ds(j, 1), :])
            return c
        jax.lax.fori_loop(0, blk, body_msg, 0)


def _head_kernel(h_ref, b_ref, w_ref, o_ref):
    logits = jnp.dot(h_ref[...] + b_ref[...], w_ref[...],
                     preferred_element_type=jnp.float32)
    logits = jnp.where(logits > 0, logits, jnp.exp(logits) - 1.0)
    m = jnp.max(logits, axis=1, keepdims=True)
    lse = jnp.log(jnp.sum(jnp.exp(logits - m), axis=1, keepdims=True)) + m
    o_ref[...] = logits - lse


def kernel(x, edge_index, lin_w, att_src, att_dst, bias, W):
    n, f_in = x.shape
    e = edge_index.shape[1]
    _, k, h = att_src.shape
    kh = k * h
    c = W.shape[1]
    n1 = n + 1

    eye = jnp.eye(k, dtype=jnp.float32)
    a_s = (att_src.reshape(k, h)[:, :, None] * eye[:, None, :]).reshape(kh, k)
    a_d = (att_dst.reshape(k, h)[:, :, None] * eye[:, None, :]).reshape(kh, k)
    asd_w = jnp.concatenate([a_s, a_d], axis=1)
    expand = jnp.repeat(eye, h, axis=1)

    loop = jnp.arange(n, dtype=edge_index.dtype)
    src = jnp.concatenate([edge_index[0], loop])
    dst = jnp.concatenate([edge_index[1], loop])
    etot = e + n
    blk = 2048 if etot >= 2048 else 8
    nb = pl.cdiv(etot, blk)
    pad = nb * blk - etot
    src = jnp.concatenate([src, jnp.full((pad,), n, src.dtype)])
    dst = jnp.concatenate([dst, jnp.full((pad,), n, dst.dtype)])
    src3 = src.reshape(nb, 1, blk)
    dst3 = dst.reshape(nb, 1, blk)

    r = min(1024, n)
    xp, asd = pl.pallas_call(
        _node_kernel,
        grid=(pl.cdiv(n, r),),
        in_specs=[
            pl.BlockSpec((r, f_in), lambda i: (i, 0)),
            pl.BlockSpec((f_in, kh), lambda i: (0, 0)),
            pl.BlockSpec((kh, 2 * k), lambda i: (0, 0)),
        ],
        out_specs=[
            pl.BlockSpec((r, kh), lambda i: (i, 0)),
            pl.BlockSpec((r, 2 * k), lambda i: (i, 0)),
        ],
        out_shape=[
            jax.ShapeDtypeStruct((n, kh), jnp.float32),
            jax.ShapeDtypeStruct((n, 2 * k), jnp.float32),
        ],
    )(x, lin_w, asd_w)

    zrow = jnp.zeros((1, kh), jnp.float32)
    xp1 = jnp.concatenate([xp, zrow])
    asd1 = jnp.concatenate([asd, jnp.zeros((1, 2 * k), jnp.float32)])

    alpha_pad, out1 = pl.pallas_call(
        functools.partial(_edge_kernel, n_heads=k),
        grid=(3, nb),
        in_specs=[
            pl.BlockSpec((1, 1, blk), lambda p, i: (i, 0, 0),
                         memory_space=pltpu.SMEM),
            pl.BlockSpec((1, 1, blk), lambda p, i: (i, 0, 0),
                         memory_space=pltpu.SMEM),
            pl.BlockSpec((n1, 2 * k), lambda p, i: (0, 0)),
            pl.BlockSpec((n1, kh), lambda p, i: (0, 0)),
            pl.BlockSpec((k, kh), lambda p, i: (0, 0)),
        ],
        out_specs=[
            pl.BlockSpec((blk, k), lambda p, i: (i, 0)),
            pl.BlockSpec((n1, kh), lambda p, i: (0, 0)),
        ],
        out_shape=[
            jax.ShapeDtypeStruct((nb * blk, k), jnp.float32),
            jax.ShapeDtypeStruct((n1, kh), jnp.float32),
        ],
        scratch_shapes=[
            pltpu.VMEM((n1, 2 * k), jnp.float32),
            pltpu.VMEM((blk, kh), jnp.float32),
        ],
        compiler_params=pltpu.CompilerParams(
            dimension_semantics=("arbitrary", "arbitrary")),
    )(src3, dst3, asd1, xp1, expand)

    alpha = alpha_pad[:etot]
    hid = out1[:n]

    logp = pl.pallas_call(
        _head_kernel,
        grid=(pl.cdiv(n, r),),
        in_specs=[
            pl.BlockSpec((r, kh), lambda i: (i, 0)),
            pl.BlockSpec((1, kh), lambda i: (0, 0)),
            pl.BlockSpec((kh, c), lambda i: (0, 0)),
        ],
        out_specs=pl.BlockSpec((r, c), lambda i: (i, 0)),
        out_shape=jax.ShapeDtypeStruct((n, c), jnp.float32),
    )(hid, bias.reshape(1, kh), W)

    return (logp, alpha)
